# baseline (speedup 1.0000x reference)
import functools

import jax
import jax.numpy as jnp
from jax import lax
from jax.experimental import pallas as pl
from jax.experimental.pallas import tpu as pltpu
from jax.experimental.pallas import tpu_sc as plsc

N = 10000
E = 320000
D = 128
EPS = 1e-5

NC = 2
NS = 16
NW = NC * NS
EPW = E // NW
CH = 128
NCHUNK = 80
EPP = NCHUNK * CH
PAD = EPP - EPW
SEG = 16
NSEG = NCHUNK // SEG
NP = 10240
RPT = NP // NS
DPT = NP // NS

BN_R = 1000
G = N // BN_R

_mesh = plsc.VectorSubcoreMesh(core_axis_name="c", subcore_axis_name="s")


@functools.partial(
    pl.kernel,
    mesh=_mesh,
    out_type=jax.ShapeDtypeStruct((NC * NP,), jnp.float32),
    scratch_types=[
        pltpu.VMEM((NCHUNK, CH), jnp.int32),
        pltpu.VMEM((CH,), jnp.float32),
        pltpu.VMEM((DPT,), jnp.float32),
        pltpu.VMEM_SHARED((NP,), jnp.float32),
    ],
)
def _deg_kernel(dst_hbm, ones_hbm, zeros_hbm, out_hbm, dstv, onesv, zerov, deg_sp):
    c = lax.axis_index("c")
    s = lax.axis_index("s")
    wid = c * NS + s
    pltpu.sync_copy(dst_hbm.at[wid], dstv)
    pltpu.sync_copy(ones_hbm, onesv)
    pltpu.sync_copy(zeros_hbm, zerov)
    pltpu.sync_copy(zerov, deg_sp.at[pl.ds(s * DPT, DPT)])
    plsc.subcore_barrier()

    def body(j, carry):
        pltpu.sync_copy(onesv, deg_sp.at[dstv.at[j]], add=True)
        return carry

    lax.fori_loop(0, NCHUNK, body, 0)
    plsc.subcore_barrier()
    pltpu.sync_copy(deg_sp.at[pl.ds(s * DPT, DPT)],
                    out_hbm.at[pl.ds(c * NP + s * DPT, DPT)])


@functools.partial(
    pl.kernel,
    mesh=_mesh,
    out_type=jax.ShapeDtypeStruct((NC, NP, D), jnp.float32),
    scratch_types=[
        pltpu.VMEM((SEG, CH), jnp.int32),
        pltpu.VMEM((SEG, CH), jnp.int32),
        pltpu.VMEM((SEG, CH), jnp.int32),
        pltpu.VMEM((SEG, CH), jnp.int32),
        pltpu.VMEM((CH, D), jnp.float32),
        pltpu.VMEM((CH, D), jnp.float32),
        pltpu.VMEM_SHARED((NP, D), jnp.float32),
        pltpu.SemaphoreType.DMA,
        pltpu.SemaphoreType.DMA,
        pltpu.SemaphoreType.DMA,
        pltpu.SemaphoreType.DMA,
        pltpu.SemaphoreType.DMA,
        pltpu.SemaphoreType.DMA,
        pltpu.SemaphoreType.DMA,
        pltpu.SemaphoreType.DMA,
    ],
)
def _scatter_kernel(hs_hbm, src_hbm, dst_hbm, zrows_hbm, out_hbm,
                    sb0, sb1, db0, db1, rows0, rows1, acc_sp,
                    ssem0, ssem1, dsem0, dsem1, rsem0, rsem1, wsem0, wsem1):
    c = lax.axis_index("c")
    s = lax.axis_index("s")
    wid = c * NS + s
    sbufs = (sb0, sb1)
    dbufs = (db0, db1)
    ssems = (ssem0, ssem1)
    dsems = (dsem0, dsem1)
    rows = (rows0, rows1)
    rsems = (rsem0, rsem1)
    wsems = (wsem0, wsem1)

    def sseg(g, p):
        return pltpu.make_async_copy(src_hbm.at[wid, pl.ds(g * SEG, SEG)],
                                     sbufs[p], ssems[p])

    def dseg(g, p):
        return pltpu.make_async_copy(dst_hbm.at[wid, pl.ds(g * SEG, SEG)],
                                     dbufs[p], dsems[p])

    def rgather(g, t, p, q):
        return pltpu.make_async_copy(hs_hbm.at[sbufs[p].at[t]], rows[q], rsems[q])

    def rscatter_start(g, t, p, q):
        pltpu.async_copy(rows[q], acc_sp.at[dbufs[p].at[t]], wsems[q], add=True)

    def rscatter_wait(g, t, p, q):
        pltpu.make_async_copy(rows[q], acc_sp.at[dbufs[p].at[t]], wsems[q]).wait()

    sseg(0, 0).start()
    dseg(0, 0).start()
    pltpu.sync_copy(zrows_hbm, rows1)
    for k in range(RPT // CH):
        pltpu.sync_copy(rows1, acc_sp.at[pl.ds(s * RPT + k * CH, CH)])
    sseg(0, 0).wait()
    rgather(0, 0, 0, 0).start()
    dseg(0, 0).wait()
    plsc.subcore_barrier()

    loc = {}
    for cidx in range(NCHUNK):
        q = cidx & 1
        g, t = divmod(cidx, SEG)
        p = g & 1
        loc[cidx] = (g, t, p, q)
        if t == 0 and g + 1 < NSEG:
            sseg(g + 1, p ^ 1).start()
            dseg(g + 1, p ^ 1).start()
        rgather(g, t, p, q).wait()
        if cidx + 1 < NCHUNK:
            ng, nt = divmod(cidx + 1, SEG)
            if nt == 0:
                sseg(ng, ng & 1).wait()
            if cidx >= 1:
                rscatter_wait(*loc[cidx - 1])
            rgather(ng, nt, ng & 1, q ^ 1).start()
        if t == 0 and g > 0:
            dseg(g, p).wait()
        rscatter_start(g, t, p, q)

    rscatter_wait(*loc[NCHUNK - 2])
    rscatter_wait(*loc[NCHUNK - 1])
    plsc.subcore_barrier()
    pltpu.sync_copy(acc_sp.at[pl.ds(s * RPT, RPT)], out_hbm.at[c, pl.ds(s * RPT, RPT)])


def _dis_body(deg_ref, dis_ref):
    d = jnp.sum(deg_ref[...], axis=0, keepdims=True) + 1.0
    dis_ref[...] = lax.rsqrt(d)


_dis_call = pl.pallas_call(
    _dis_body,
    grid=(1,),
    in_specs=[pl.BlockSpec((NC, NP), lambda i: (0, 0))],
    out_specs=pl.BlockSpec((1, NP), lambda i: (0, 0)),
    out_shape=jax.ShapeDtypeStruct((1, NP), jnp.float32),
)


def _mm_first_body(x_ref, w_ref, dis_ref, out_ref):
    h = jnp.dot(x_ref[...], w_ref[...], preferred_element_type=jnp.float32)
    out_ref[...] = h * dis_ref[...]


_mm_first = pl.pallas_call(
    _mm_first_body,
    grid=(G,),
    in_specs=[
        pl.BlockSpec((BN_R, D), lambda i: (i, 0)),
        pl.BlockSpec((D, D), lambda i: (0, 0)),
        pl.BlockSpec((BN_R, 1), lambda i: (i, 0)),
    ],
    out_specs=pl.BlockSpec((BN_R, D), lambda i: (i, 0)),
    out_shape=jax.ShapeDtypeStruct((N, D), jnp.float32),
)


def _fused_body(msg_ref, hs_ref, dis_ref, b_ref, g_ref, bt_ref, w_ref,
                out_ref, t_scr, s_scr, q_scr, a_scr, bb_scr):
    ph = pl.program_id(0)
    i = pl.program_id(1)

    @pl.when(ph == 0)
    def _():
        t = (msg_ref[0] + msg_ref[1] + hs_ref[...]) * dis_ref[...] + b_ref[...]
        t_scr[pl.ds(i * BN_R, BN_R), :] = t
        ps = jnp.sum(t, axis=0, keepdims=True)
        pq = jnp.sum(t * t, axis=0, keepdims=True)

        @pl.when(i == 0)
        def _():
            s_scr[...] = ps
            q_scr[...] = pq

        @pl.when(i != 0)
        def _():
            s_scr[...] += ps
            q_scr[...] += pq

    @pl.when(ph == 1)
    def _():
        @pl.when(i == 0)
        def _():
            mean = s_scr[...] * (1.0 / N)
            var = q_scr[...] * (1.0 / N) - mean * mean
            a = g_ref[...] * lax.rsqrt(var + EPS)
            a_scr[...] = a
            bb_scr[...] = bt_ref[...] - mean * a

        t = t_scr[pl.ds(i * BN_R, BN_R), :]
        u = jnp.maximum(t * a_scr[...] + bb_scr[...], 0.0)
        h = jnp.dot(u, w_ref[...], preferred_element_type=jnp.float32)
        out_ref[...] = h * dis_ref[...]


_fused = pl.pallas_call(
    _fused_body,
    grid=(2, G),
    in_specs=[
        pl.BlockSpec((NC, BN_R, D), lambda ph, i: (0, i * (1 - ph), 0)),
        pl.BlockSpec((BN_R, D), lambda ph, i: (i * (1 - ph), 0)),
        pl.BlockSpec((BN_R, 1), lambda ph, i: (i, 0)),
        pl.BlockSpec((1, D), lambda ph, i: (0, 0)),
        pl.BlockSpec((1, D), lambda ph, i: (0, 0)),
        pl.BlockSpec((1, D), lambda ph, i: (0, 0)),
        pl.BlockSpec((D, D), lambda ph, i: (0, 0)),
    ],
    out_specs=pl.BlockSpec((BN_R, D), lambda ph, i: (i, 0)),
    out_shape=jax.ShapeDtypeStruct((N, D), jnp.float32),
    scratch_shapes=[
        pltpu.VMEM((N, D), jnp.float32),
        pltpu.VMEM((1, D), jnp.float32),
        pltpu.VMEM((1, D), jnp.float32),
        pltpu.VMEM((1, D), jnp.float32),
        pltpu.VMEM((1, D), jnp.float32),
    ],
)


def _comb_last_body(msg_ref, hs_ref, dis_ref, b_ref, out_ref):
    out_ref[...] = (msg_ref[0] + msg_ref[1] + hs_ref[...]) * dis_ref[...] + b_ref[...]


_comb_last = pl.pallas_call(
    _comb_last_body,
    grid=(G,),
    in_specs=[
        pl.BlockSpec((NC, BN_R, D), lambda i: (0, i, 0)),
        pl.BlockSpec((BN_R, D), lambda i: (i, 0)),
        pl.BlockSpec((BN_R, 1), lambda i: (i, 0)),
        pl.BlockSpec((1, D), lambda i: (0, 0)),
    ],
    out_specs=pl.BlockSpec((BN_R, D), lambda i: (i, 0)),
    out_shape=jax.ShapeDtypeStruct((N, D), jnp.float32),
)


def kernel(x, adj_t, W0, b0, W1, b1, W2, b2, g0, beta0, g1, beta1):
    w = jnp.arange(NW, dtype=jnp.int32)[:, None]
    j = jnp.arange(PAD, dtype=jnp.int32)[None, :]
    fsrc = (w * 131 + j * 17) % N
    fdst = N + (w * 7 + j % 7)
    src = jnp.concatenate([adj_t[0].reshape(NW, EPW), fsrc], axis=1)
    dst = jnp.concatenate([adj_t[1].reshape(NW, EPW), fdst], axis=1)
    src = src.reshape(NW, NCHUNK, CH)
    dst = dst.reshape(NW, NCHUNK, CH)
    ones = jnp.ones((CH,), jnp.float32)
    zeros1 = jnp.zeros((DPT,), jnp.float32)
    zrows = jnp.zeros((CH, D), jnp.float32)

    deg = _deg_kernel(dst, ones, zeros1).reshape(NC, NP)
    dis_row = _dis_call(deg)
    dis = dis_row.reshape(NP)[:N].reshape(N, 1)

    b0r = b0.reshape(1, D)
    b1r = b1.reshape(1, D)
    b2r = b2.reshape(1, D)
    g0r = g0.reshape(1, D)
    g1r = g1.reshape(1, D)
    bt0r = beta0.reshape(1, D)
    bt1r = beta1.reshape(1, D)

    hs0 = _mm_first(x, W0, dis)
    msg0 = _scatter_kernel(hs0, src, dst, zrows)
    hs1 = _fused(msg0, hs0, dis, b0r, g0r, bt0r, W1)
    msg1 = _scatter_kernel(hs1, src, dst, zrows)
    hs2 = _fused(msg1, hs1, dis, b1r, g1r, bt1r, W2)
    msg2 = _scatter_kernel(hs2, src, dst, zrows)
    return _comb_last(msg2, hs2, dis, b2r)


# speedup vs baseline: 1.0030x; 1.0030x over previous
import functools

import jax
import jax.numpy as jnp
from jax import lax
from jax.experimental import pallas as pl
from jax.experimental.pallas import tpu as pltpu
from jax.experimental.pallas import tpu_sc as plsc

N = 10000
E = 320000
D = 128
EPS = 1e-5

NC = 2
NS = 16
NW = NC * NS
EPW = E // NW
CH = 125
NCHUNK = 80
SEG = 16
NSEG = NCHUNK // SEG
NP = 10240
RPT = NP // NS
DPT = NP // NS

BN_R = 1000
G = N // BN_R

_mesh = plsc.VectorSubcoreMesh(core_axis_name="c", subcore_axis_name="s")


@functools.partial(
    pl.kernel,
    mesh=_mesh,
    out_type=jax.ShapeDtypeStruct((NC * NP,), jnp.float32),
    scratch_types=[
        pltpu.VMEM((NCHUNK, CH), jnp.int32),
        pltpu.VMEM((CH,), jnp.float32),
        pltpu.VMEM((DPT,), jnp.float32),
        pltpu.VMEM_SHARED((NP,), jnp.float32),
    ],
)
def _deg_kernel(dst_hbm, ones_hbm, zeros_hbm, out_hbm, dstv, onesv, zerov, deg_sp):
    c = lax.axis_index("c")
    s = lax.axis_index("s")
    wid = c * NS + s
    pltpu.sync_copy(dst_hbm.at[wid], dstv)
    pltpu.sync_copy(ones_hbm, onesv)
    pltpu.sync_copy(zeros_hbm, zerov)
    pltpu.sync_copy(zerov, deg_sp.at[pl.ds(s * DPT, DPT)])
    plsc.subcore_barrier()

    def body(j, carry):
        pltpu.sync_copy(onesv, deg_sp.at[dstv.at[j]], add=True)
        return carry

    lax.fori_loop(0, NCHUNK, body, 0)
    plsc.subcore_barrier()
    pltpu.sync_copy(deg_sp.at[pl.ds(s * DPT, DPT)],
                    out_hbm.at[pl.ds(c * NP + s * DPT, DPT)])


@functools.partial(
    pl.kernel,
    mesh=_mesh,
    out_type=jax.ShapeDtypeStruct((NC, NP, D), jnp.float32),
    scratch_types=[
        pltpu.VMEM((SEG, CH), jnp.int32),
        pltpu.VMEM((SEG, CH), jnp.int32),
        pltpu.VMEM((SEG, CH), jnp.int32),
        pltpu.VMEM((SEG, CH), jnp.int32),
        pltpu.VMEM((CH, D), jnp.float32),
        pltpu.VMEM((CH, D), jnp.float32),
        pltpu.VMEM_SHARED((NP, D), jnp.float32),
        pltpu.SemaphoreType.DMA,
        pltpu.SemaphoreType.DMA,
        pltpu.SemaphoreType.DMA,
        pltpu.SemaphoreType.DMA,
        pltpu.SemaphoreType.DMA,
        pltpu.SemaphoreType.DMA,
        pltpu.SemaphoreType.DMA,
        pltpu.SemaphoreType.DMA,
    ],
)
def _scatter_kernel(hs_hbm, src_hbm, dst_hbm, zrows_hbm, out_hbm,
                    sb0, sb1, db0, db1, rows0, rows1, acc_sp,
                    ssem0, ssem1, dsem0, dsem1, rsem0, rsem1, wsem0, wsem1):
    c = lax.axis_index("c")
    s = lax.axis_index("s")
    wid = c * NS + s
    sbufs = (sb0, sb1)
    dbufs = (db0, db1)
    ssems = (ssem0, ssem1)
    dsems = (dsem0, dsem1)
    rows = (rows0, rows1)
    rsems = (rsem0, rsem1)
    wsems = (wsem0, wsem1)

    def sseg(g, p):
        return pltpu.make_async_copy(src_hbm.at[wid, pl.ds(g * SEG, SEG)],
                                     sbufs[p], ssems[p])

    def dseg(g, p):
        return pltpu.make_async_copy(dst_hbm.at[wid, pl.ds(g * SEG, SEG)],
                                     dbufs[p], dsems[p])

    def rgather(g, t, p, q):
        return pltpu.make_async_copy(hs_hbm.at[sbufs[p].at[t]], rows[q], rsems[q])

    def rscatter_start(g, t, p, q):
        pltpu.async_copy(rows[q], acc_sp.at[dbufs[p].at[t]], wsems[q], add=True)

    def rscatter_wait(g, t, p, q):
        pltpu.make_async_copy(rows[q], acc_sp.at[dbufs[p].at[t]], wsems[q]).wait()

    sseg(0, 0).start()
    dseg(0, 0).start()
    pltpu.sync_copy(zrows_hbm, rows1)
    off = 0
    while off < RPT:
        step = min(CH, RPT - off)
        pltpu.sync_copy(rows1.at[pl.ds(0, step)],
                        acc_sp.at[pl.ds(s * RPT + off, step)])
        off += step
    sseg(0, 0).wait()
    rgather(0, 0, 0, 0).start()
    dseg(0, 0).wait()
    plsc.subcore_barrier()

    loc = {}
    for cidx in range(NCHUNK):
        q = cidx & 1
        g, t = divmod(cidx, SEG)
        p = g & 1
        loc[cidx] = (g, t, p, q)
        if t == 0 and g + 1 < NSEG:
            sseg(g + 1, p ^ 1).start()
            dseg(g + 1, p ^ 1).start()
        rgather(g, t, p, q).wait()
        if cidx + 1 < NCHUNK:
            ng, nt = divmod(cidx + 1, SEG)
            if nt == 0:
                sseg(ng, ng & 1).wait()
            if cidx >= 1:
                rscatter_wait(*loc[cidx - 1])
            rgather(ng, nt, ng & 1, q ^ 1).start()
        if t == 0 and g > 0:
            dseg(g, p).wait()
        rscatter_start(g, t, p, q)

    rscatter_wait(*loc[NCHUNK - 2])
    rscatter_wait(*loc[NCHUNK - 1])
    plsc.subcore_barrier()
    pltpu.sync_copy(acc_sp.at[pl.ds(s * RPT, RPT)], out_hbm.at[c, pl.ds(s * RPT, RPT)])


def _dis_body(deg_ref, dis_ref):
    d = jnp.sum(deg_ref[...], axis=0, keepdims=True) + 1.0
    dis_ref[...] = lax.rsqrt(d)


_dis_call = pl.pallas_call(
    _dis_body,
    grid=(1,),
    in_specs=[pl.BlockSpec((NC, NP), lambda i: (0, 0))],
    out_specs=pl.BlockSpec((1, NP), lambda i: (0, 0)),
    out_shape=jax.ShapeDtypeStruct((1, NP), jnp.float32),
)


def _mm_first_body(x_ref, w_ref, dis_ref, out_ref):
    h = jnp.dot(x_ref[...], w_ref[...], preferred_element_type=jnp.float32)
    out_ref[...] = h * dis_ref[...]


_mm_first = pl.pallas_call(
    _mm_first_body,
    grid=(G,),
    in_specs=[
        pl.BlockSpec((BN_R, D), lambda i: (i, 0)),
        pl.BlockSpec((D, D), lambda i: (0, 0)),
        pl.BlockSpec((BN_R, 1), lambda i: (i, 0)),
    ],
    out_specs=pl.BlockSpec((BN_R, D), lambda i: (i, 0)),
    out_shape=jax.ShapeDtypeStruct((N, D), jnp.float32),
)


def _fused_body(msg_ref, hs_ref, dis_ref, b_ref, g_ref, bt_ref, w_ref,
                out_ref, t_scr, s_scr, q_scr, a_scr, bb_scr):
    ph = pl.program_id(0)
    i = pl.program_id(1)

    @pl.when(ph == 0)
    def _():
        t = (msg_ref[0] + msg_ref[1] + hs_ref[...]) * dis_ref[...] + b_ref[...]
        t_scr[pl.ds(i * BN_R, BN_R), :] = t
        ps = jnp.sum(t, axis=0, keepdims=True)
        pq = jnp.sum(t * t, axis=0, keepdims=True)

        @pl.when(i == 0)
        def _():
            s_scr[...] = ps
            q_scr[...] = pq

        @pl.when(i != 0)
        def _():
            s_scr[...] += ps
            q_scr[...] += pq

    @pl.when(ph == 1)
    def _():
        @pl.when(i == 0)
        def _():
            mean = s_scr[...] * (1.0 / N)
            var = q_scr[...] * (1.0 / N) - mean * mean
            a = g_ref[...] * lax.rsqrt(var + EPS)
            a_scr[...] = a
            bb_scr[...] = bt_ref[...] - mean * a

        t = t_scr[pl.ds(i * BN_R, BN_R), :]
        u = jnp.maximum(t * a_scr[...] + bb_scr[...], 0.0)
        h = jnp.dot(u, w_ref[...], preferred_element_type=jnp.float32)
        out_ref[...] = h * dis_ref[...]


_fused = pl.pallas_call(
    _fused_body,
    grid=(2, G),
    in_specs=[
        pl.BlockSpec((NC, BN_R, D), lambda ph, i: (0, i * (1 - ph), 0)),
        pl.BlockSpec((BN_R, D), lambda ph, i: (i * (1 - ph), 0)),
        pl.BlockSpec((BN_R, 1), lambda ph, i: (i, 0)),
        pl.BlockSpec((1, D), lambda ph, i: (0, 0)),
        pl.BlockSpec((1, D), lambda ph, i: (0, 0)),
        pl.BlockSpec((1, D), lambda ph, i: (0, 0)),
        pl.BlockSpec((D, D), lambda ph, i: (0, 0)),
    ],
    out_specs=pl.BlockSpec((BN_R, D), lambda ph, i: (i, 0)),
    out_shape=jax.ShapeDtypeStruct((N, D), jnp.float32),
    scratch_shapes=[
        pltpu.VMEM((N, D), jnp.float32),
        pltpu.VMEM((1, D), jnp.float32),
        pltpu.VMEM((1, D), jnp.float32),
        pltpu.VMEM((1, D), jnp.float32),
        pltpu.VMEM((1, D), jnp.float32),
    ],
)


def _comb_last_body(msg_ref, hs_ref, dis_ref, b_ref, out_ref):
    out_ref[...] = ((msg_ref[0] + msg_ref[1] + hs_ref[...]) * dis_ref[...]
                    + b_ref[...])


_comb_last = pl.pallas_call(
    _comb_last_body,
    grid=(G,),
    in_specs=[
        pl.BlockSpec((NC, BN_R, D), lambda i: (0, i, 0)),
        pl.BlockSpec((BN_R, D), lambda i: (i, 0)),
        pl.BlockSpec((BN_R, 1), lambda i: (i, 0)),
        pl.BlockSpec((1, D), lambda i: (0, 0)),
    ],
    out_specs=pl.BlockSpec((BN_R, D), lambda i: (i, 0)),
    out_shape=jax.ShapeDtypeStruct((N, D), jnp.float32),
)


def kernel(x, adj_t, W0, b0, W1, b1, W2, b2, g0, beta0, g1, beta1):
    src = adj_t[0].reshape(NW, NCHUNK, CH)
    dst = adj_t[1].reshape(NW, NCHUNK, CH)
    ones = jnp.ones((CH,), jnp.float32)
    zeros1 = jnp.zeros((DPT,), jnp.float32)
    zrows = jnp.zeros((CH, D), jnp.float32)

    deg = _deg_kernel(dst, ones, zeros1).reshape(NC, NP)
    dis_row = _dis_call(deg)
    dis = dis_row.reshape(NP)[:N].reshape(N, 1)

    b0r = b0.reshape(1, D)
    b1r = b1.reshape(1, D)
    b2r = b2.reshape(1, D)
    g0r = g0.reshape(1, D)
    g1r = g1.reshape(1, D)
    bt0r = beta0.reshape(1, D)
    bt1r = beta1.reshape(1, D)

    hs0 = _mm_first(x, W0, dis)
    msg0 = _scatter_kernel(hs0, src, dst, zrows)
    hs1 = _fused(msg0, hs0, dis, b0r, g0r, bt0r, W1)
    msg1 = _scatter_kernel(hs1, src, dst, zrows)
    hs2 = _fused(msg1, hs1, dis, b1r, g1r, bt1r, W2)
    msg2 = _scatter_kernel(hs2, src, dst, zrows)
    return _comb_last(msg2, hs2, dis, b2r)
